# speedup vs baseline: 1.3257x; 1.3257x over previous
import functools

import jax
import jax.numpy as jnp
from jax import lax
from jax.experimental import pallas as pl
from jax.experimental.pallas import tpu as pltpu
from jax.experimental.pallas import tpu_sc as plsc

DIM = 768
BATCH = 4096
SEQ = 50
B = BATCH * SEQ

NC, NS = 2, 16
NW = NC * NS
B_PER_W = B // NW
CHUNK = 128
N_CHUNKS = B_PER_W // CHUNK

_mesh = plsc.VectorSubcoreMesh(core_axis_name="c", subcore_axis_name="s")


@functools.partial(
    pl.kernel,
    out_type=jax.ShapeDtypeStruct((B, DIM), jnp.float32),
    mesh=_mesh,
    scratch_types=[
        pltpu.VMEM((B_PER_W,), jnp.int32),
        pltpu.VMEM((CHUNK, DIM), jnp.float32),
        pltpu.SemaphoreType.DMA,
    ],
)
def _embedding_gather(idx_hbm, table_hbm, out_hbm, idx_v, rows_v, sem):
    wid = lax.axis_index("s") * NC + lax.axis_index("c")
    base = wid * B_PER_W
    pltpu.sync_copy(idx_hbm.at[pl.ds(base, B_PER_W)], idx_v)

    def chunk_body(i, carry):
        off = i * CHUNK
        pltpu.async_copy(
            table_hbm.at[idx_v.at[pl.ds(off, CHUNK)]], rows_v, sem
        ).wait()
        pltpu.sync_copy(rows_v, out_hbm.at[pl.ds(base + off, CHUNK)])
        return carry

    lax.fori_loop(0, N_CHUNKS, chunk_body, 0)


def kernel(news_batch, table):
    flat_idx = news_batch.reshape(B)
    out = _embedding_gather(flat_idx, table)
    return out.reshape(BATCH, SEQ, DIM)


# speedup vs baseline: 1.3345x; 1.0066x over previous
import functools

import jax
import jax.numpy as jnp
from jax import lax
from jax.experimental import pallas as pl
from jax.experimental.pallas import tpu as pltpu
from jax.experimental.pallas import tpu_sc as plsc

DIM = 768
BATCH = 4096
SEQ = 50
B = BATCH * SEQ

NC, NS = 2, 16
NW = NC * NS
B_PER_W = B // NW
CHUNK = 32
NBUF = 4
LOOKAHEAD = 2
N_CHUNKS = B_PER_W // CHUNK
N_OUTER = N_CHUNKS // NBUF

_mesh = plsc.VectorSubcoreMesh(core_axis_name="c", subcore_axis_name="s")


@functools.partial(
    pl.kernel,
    out_type=jax.ShapeDtypeStruct((B, DIM), jnp.float32),
    mesh=_mesh,
    scratch_types=[
        pltpu.VMEM((B_PER_W,), jnp.int32),
        [pltpu.VMEM((CHUNK, DIM), jnp.float32) for _ in range(NBUF)],
        [pltpu.SemaphoreType.DMA for _ in range(NBUF)],
        [pltpu.SemaphoreType.DMA for _ in range(NBUF)],
    ],
)
def _embedding_gather(idx_hbm, table_hbm, out_hbm, idx_v, rows, semg, semo):
    wid = lax.axis_index("s") * NC + lax.axis_index("c")
    base = wid * B_PER_W
    pltpu.sync_copy(idx_hbm.at[pl.ds(base, B_PER_W)], idx_v)

    def gather(c, b):
        pltpu.async_copy(
            table_hbm.at[idx_v.at[pl.ds(c * CHUNK, CHUNK)]], rows[b], semg[b]
        )

    def gather_wait(b):
        pltpu.make_async_copy(
            table_hbm.at[idx_v.at[pl.ds(0, CHUNK)]], rows[b], semg[b]
        ).wait()

    def out_copy(c, b):
        pltpu.async_copy(rows[b], out_hbm.at[pl.ds(base + c * CHUNK, CHUNK)], semo[b])

    def out_drain(b):
        pltpu.make_async_copy(
            rows[b], out_hbm.at[pl.ds(base, CHUNK)], semo[b]
        ).wait()

    gather(0, 0)
    gather(1, 1)

    def outer(i, carry):
        for b in range(NBUF):
            c = i * NBUF + b
            f = c + LOOKAHEAD
            bf = (b + LOOKAHEAD) % NBUF
            issue = (i < N_OUTER - 1) if b >= NBUF - LOOKAHEAD else True
            drain_needed = i >= 1 if b < LOOKAHEAD else True
            if issue is True and drain_needed is True:
                out_drain(bf)
                gather(f, bf)
            elif issue is True:
                @pl.when(i >= 1)
                def _():
                    out_drain(bf)

                gather(f, bf)
            else:
                @pl.when(i < N_OUTER - 1)
                def _():
                    out_drain(bf)
                    gather(f, bf)

            gather_wait(b)
            out_copy(c, b)
        return carry

    lax.fori_loop(0, N_OUTER, outer, 0)

    for b in range(NBUF):
        out_drain(b)


def kernel(news_batch, table):
    flat_idx = news_batch.reshape(B)
    out = _embedding_gather(flat_idx, table)
    return out.reshape(BATCH, SEQ, DIM)


# speedup vs baseline: 4.2805x; 3.2076x over previous
import functools

import jax
import jax.numpy as jnp
from jax import lax
from jax.experimental import pallas as pl
from jax.experimental.pallas import tpu as pltpu
from jax.experimental.pallas import tpu_sc as plsc

DIM = 768
BATCH = 4096
SEQ = 50
B = BATCH * SEQ

NC, NS = 2, 16
NW = NC * NS
B_PER_W = B // NW
CHUNK = 32
NBUF = 4
LOOKAHEAD = 2
N_CHUNKS = B_PER_W // CHUNK
N_OUTER = N_CHUNKS // NBUF

_mesh = plsc.VectorSubcoreMesh(core_axis_name="c", subcore_axis_name="s")


@functools.partial(
    pl.kernel,
    out_type=jax.ShapeDtypeStruct((B, DIM), jnp.float32),
    mesh=_mesh,
    scratch_types=[
        pltpu.VMEM((B_PER_W,), jnp.int32),
        [pltpu.VMEM((CHUNK, DIM), jnp.float32) for _ in range(NBUF)],
        [pltpu.SemaphoreType.DMA for _ in range(NBUF)],
        [pltpu.SemaphoreType.DMA for _ in range(NBUF)],
    ],
)
def _embedding_gather(idx_hbm, table_hbm, out_hbm, idx_v, rows, semg, semo):
    wid = lax.axis_index("s") * NC + lax.axis_index("c")
    base = wid * B_PER_W
    pltpu.sync_copy(idx_hbm.at[pl.ds(base, B_PER_W)], idx_v)

    def gather(c, b):
        pltpu.async_copy(
            table_hbm.at[idx_v.at[pl.ds(c * CHUNK, CHUNK)]], rows[b], semg[b]
        )

    def gather_wait(b):
        pltpu.make_async_copy(
            table_hbm.at[idx_v.at[pl.ds(0, CHUNK)]], rows[b], semg[b]
        ).wait()

    def out_copy(c, b):
        pltpu.async_copy(rows[b], out_hbm.at[pl.ds(base + c * CHUNK, CHUNK)], semo[b])

    def out_drain(b):
        pltpu.make_async_copy(
            rows[b], out_hbm.at[pl.ds(base, CHUNK)], semo[b]
        ).wait()

    gather(0, 0)
    gather(1, 1)

    def outer(i, carry):
        for b in range(NBUF):
            c = i * NBUF + b
            f = c + LOOKAHEAD
            bf = (b + LOOKAHEAD) % NBUF
            issue = (i < N_OUTER - 1) if b >= NBUF - LOOKAHEAD else True
            drain_needed = i >= 1 if b < LOOKAHEAD else True
            if issue is True and drain_needed is True:
                out_drain(bf)
                gather(f, bf)
            elif issue is True:
                @pl.when(i >= 1)
                def _():
                    out_drain(bf)

                gather(f, bf)
            else:
                @pl.when(i < N_OUTER - 1)
                def _():
                    out_drain(bf)
                    gather(f, bf)

            gather_wait(b)
            out_copy(c, b)
        return carry

    lax.fori_loop(0, N_OUTER, outer, 0)

    for b in range(NBUF):
        out_drain(b)


def kernel(news_batch, table):
    flat_idx = news_batch.T.reshape(B)
    out = _embedding_gather(flat_idx, table)
    return out.reshape(SEQ, BATCH, DIM).transpose(1, 0, 2)
